# Initial kernel scaffold; baseline (speedup 1.0000x reference)
import jax
import jax.numpy as jnp
from jax.experimental import pallas as pl


def kernel(x, edge_index, edge_attr, W1, b1, W2, b2, W3, b3, g0_Wl, g0_bl, g0_Wr, g0_br, g0_att, g0_We, g0_bias, g1_Wl, g1_bl, g1_Wr, g1_br, g1_att, g1_We, g1_bias):
    raise NotImplementedError("write your pallas kernel here")


# speedup vs baseline: 5.3303x; 5.3303x over previous
import functools

import jax
import jax.numpy as jnp
from jax import lax
from jax.experimental import pallas as pl
from jax.experimental.pallas import tpu as pltpu
from jax.experimental.pallas import tpu_sc as plsc

NN = 10000
NE = 320000
D = 128
H1, H2 = 512, 256

NCORE = 2
NSUB = 16
NW = NCORE * NSUB
EPW = NE // NW
CH = 80
NCHUNK = EPW // CH
NN_PAD = 10240
RPT = NN_PAD // NSUB
DEN_W = 16

BM = 400
GRID = NN // BM

_f32 = jnp.float32
_i32 = jnp.int32


def _relu6(v):
    return jnp.clip(v, 0.0, 6.0)


def _embed_body(x_ref, W1_ref, b1_ref, W2_ref, b2_ref, W3_ref, b3_ref,
                Wl_ref, bl_ref, Wr_ref, br_ref, xl_ref, xr_ref):
    h = jnp.dot(x_ref[...], W1_ref[...], preferred_element_type=_f32) + b1_ref[...]
    h = _relu6(h)
    h = jnp.dot(h, W2_ref[...], preferred_element_type=_f32) + b2_ref[...]
    h = _relu6(h)
    h = jnp.dot(h, W3_ref[...], preferred_element_type=_f32) + b3_ref[...]
    xl_ref[...] = jnp.dot(h, Wl_ref[...], preferred_element_type=_f32) + bl_ref[...]
    xr_ref[...] = jnp.dot(h, Wr_ref[...], preferred_element_type=_f32) + br_ref[...]


def _embed(x, W1, b1, W2, b2, W3, b3, Wl, bl, Wr, br):
    full = lambda a, b: pl.BlockSpec((a, b), lambda i: (0, 0))
    return pl.pallas_call(
        _embed_body,
        grid=(GRID,),
        in_specs=[
            pl.BlockSpec((BM, D), lambda i: (i, 0)),
            full(D, H1), full(1, H1),
            full(H1, H2), full(1, H2),
            full(H2, D), full(1, D),
            full(D, D), full(1, D),
            full(D, D), full(1, D),
        ],
        out_specs=[pl.BlockSpec((BM, D), lambda i: (i, 0)),
                   pl.BlockSpec((BM, D), lambda i: (i, 0))],
        out_shape=[jax.ShapeDtypeStruct((NN, D), _f32),
                   jax.ShapeDtypeStruct((NN, D), _f32)],
    )(x, W1, b1.reshape(1, -1), W2, b2.reshape(1, -1), W3, b3.reshape(1, -1),
      Wl, bl.reshape(1, -1), Wr, br.reshape(1, -1))


def _combine_proj_body(n_ref, d_ref, bias_ref, Wl_ref, bl_ref, Wr_ref, br_ref,
                       xl_ref, xr_ref):
    num = n_ref[0] + n_ref[1]
    den = jnp.sum(d_ref[...], axis=1)
    h = num / (den + 1e-16)[:, None] + bias_ref[...]
    h = _relu6(h)
    xl_ref[...] = jnp.dot(h, Wl_ref[...], preferred_element_type=_f32) + bl_ref[...]
    xr_ref[...] = jnp.dot(h, Wr_ref[...], preferred_element_type=_f32) + br_ref[...]


def _combine_proj(accN, accD, bias, Wl, bl, Wr, br):
    full = lambda a, b: pl.BlockSpec((a, b), lambda i: (0, 0))
    return pl.pallas_call(
        _combine_proj_body,
        grid=(GRID,),
        in_specs=[
            pl.BlockSpec((2, BM, D), lambda i: (0, i, 0)),
            pl.BlockSpec((BM, 2 * NSUB), lambda i: (i, 0)),
            full(1, D),
            full(D, D), full(1, D),
            full(D, D), full(1, D),
        ],
        out_specs=[pl.BlockSpec((BM, D), lambda i: (i, 0)),
                   pl.BlockSpec((BM, D), lambda i: (i, 0))],
        out_shape=[jax.ShapeDtypeStruct((NN, D), _f32),
                   jax.ShapeDtypeStruct((NN, D), _f32)],
    )(accN, accD, bias.reshape(1, -1), Wl, bl.reshape(1, -1), Wr, br.reshape(1, -1))


def _combine_final_body(n_ref, d_ref, bias_ref, out_ref):
    num = n_ref[0] + n_ref[1]
    den = jnp.sum(d_ref[...], axis=1)
    h = num / (den + 1e-16)[:, None] + bias_ref[...]
    out_ref[...] = _relu6(h)


def _combine_final(accN, accD, bias):
    full = lambda a, b: pl.BlockSpec((a, b), lambda i: (0, 0))
    return pl.pallas_call(
        _combine_final_body,
        grid=(GRID,),
        in_specs=[
            pl.BlockSpec((2, BM, D), lambda i: (0, i, 0)),
            pl.BlockSpec((BM, 2 * NSUB), lambda i: (i, 0)),
            full(1, D),
        ],
        out_specs=pl.BlockSpec((BM, D), lambda i: (i, 0)),
        out_shape=jax.ShapeDtypeStruct((NN, D), _f32),
    )(accN, accD, bias.reshape(1, -1))


def _gat_edge(xl, xr, src, dst, ea, We, att):
    mesh = plsc.VectorSubcoreMesh(core_axis_name="c", subcore_axis_name="s")

    @functools.partial(
        pl.kernel,
        mesh=mesh,
        compiler_params=pltpu.CompilerParams(needs_layout_passes=False),
        out_type=(jax.ShapeDtypeStruct((NCORE, NN_PAD, D), _f32),
                  jax.ShapeDtypeStruct((NCORE, NSUB, NN_PAD), _f32)),
        scratch_types=[
            pltpu.VMEM((CH,), _i32),
            pltpu.VMEM((CH,), _i32),
            pltpu.VMEM((CH,), _f32),
            pltpu.VMEM((CH, D), _f32),
            pltpu.VMEM((CH, D), _f32),
            pltpu.VMEM((NN_PAD,), _f32),
            pltpu.VMEM((D,), _f32),
            pltpu.VMEM((D,), _f32),
            pltpu.VMEM_SHARED((NN_PAD, D), _f32),
            pltpu.SemaphoreType.DMA,
            pltpu.SemaphoreType.DMA,
        ],
    )
    def k(xl_h, xr_h, src_h, dst_h, ea_h, We_h, att_h, outN, outD,
          srcv, dstv, eav, A, B, denv, Wev, attv, Nsh,
          sem0, sem1):
        c = lax.axis_index("c")
        s = lax.axis_index("s")
        w = c * NSUB + s

        pltpu.sync_copy(We_h, Wev)
        pltpu.sync_copy(att_h, attv)
        Wv = [Wev[16 * j:16 * (j + 1)] for j in range(D // 16)]
        Av = [attv[16 * j:16 * (j + 1)] for j in range(D // 16)]

        zero16 = jnp.zeros((16,), _f32)
        for e in range(CH):
            for j in range(D // 16):
                A[e, 16 * j:16 * (j + 1)] = zero16

        def zden(i, carry):
            denv[pl.ds(i * 16, 16)] = zero16
            return carry

        lax.fori_loop(0, NN_PAD // 16, zden, 0)
        r0 = s * RPT
        for r in range(RPT // CH):
            pltpu.sync_copy(A, Nsh.at[pl.ds(r0 + r * CH, CH)])
        plsc.subcore_barrier()

        rows16 = lax.iota(_i32, 16)

        def chunk(i, carry):
            base = w * EPW + i * CH
            pltpu.sync_copy(src_h.at[pl.ds(base, CH)], srcv)
            pltpu.sync_copy(dst_h.at[pl.ds(base, CH)], dstv)
            pltpu.sync_copy(ea_h.at[pl.ds(base, CH)], eav)
            ca = pltpu.async_copy(xl_h.at[srcv], A, sem0)
            cb = pltpu.async_copy(xr_h.at[dstv], B, sem1)
            ca.wait()
            cb.wait()
            for g in range(CH // 16):
                ea_g = eav[16 * g:16 * (g + 1)]
                lvec = zero16
                for t in range(16):
                    e = g * 16 + t
                    ea_e = ea_g[t]
                    acc = zero16
                    for j in range(D // 16):
                        u = A[e, 16 * j:16 * (j + 1)] + B[e, 16 * j:16 * (j + 1)] + ea_e * Wv[j]
                        u = jnp.maximum(u, 0.2 * u)
                        acc = acc + Av[j] * u
                    lvec = jnp.where(rows16 == t, jnp.sum(acc), lvec)
                exvec = jnp.exp(lvec)
                plsc.addupdate_scatter(denv, [dstv[16 * g:16 * (g + 1)]], exvec)
                for t in range(16):
                    e = g * 16 + t
                    xs = exvec[t]
                    for j in range(D // 16):
                        A[e, 16 * j:16 * (j + 1)] = A[e, 16 * j:16 * (j + 1)] * xs
            pltpu.sync_copy(A, Nsh.at[dstv], add=True)
            return carry

        lax.fori_loop(0, NCHUNK, chunk, 0)
        pltpu.sync_copy(denv, outD.at[c, s])
        plsc.subcore_barrier()

        for r in range(RPT // CH):
            pltpu.sync_copy(Nsh.at[pl.ds(r0 + r * CH, CH)], A)
            pltpu.sync_copy(A, outN.at[c, pl.ds(r0 + r * CH, CH)])

    return k(xl, xr, src, dst, ea, We, att)


def kernel(x, edge_index, edge_attr, W1, b1, W2, b2, W3, b3,
           g0_Wl, g0_bl, g0_Wr, g0_br, g0_att, g0_We, g0_bias,
           g1_Wl, g1_bl, g1_Wr, g1_br, g1_att, g1_We, g1_bias):
    src = edge_index[0].astype(_i32)
    dst = edge_index[1].astype(_i32)
    ea = edge_attr.reshape(NE)

    xl0, xr0 = _embed(x, W1, b1, W2, b2, W3, b3, g0_Wl, g0_bl, g0_Wr, g0_br)
    N0, D0 = _gat_edge(xl0, xr0, src, dst, ea, g0_We.reshape(D), g0_att)
    d0 = D0.reshape(2 * NSUB, NN_PAD)[:, :NN].T
    xl1, xr1 = _combine_proj(N0[:, :NN], d0, g0_bias,
                             g1_Wl, g1_bl, g1_Wr, g1_br)
    N1, D1 = _gat_edge(xl1, xr1, src, dst, ea, g1_We.reshape(D), g1_att)
    d1 = D1.reshape(2 * NSUB, NN_PAD)[:, :NN].T
    return _combine_final(N1[:, :NN], d1, g1_bias)


# speedup vs baseline: 7.8351x; 1.4699x over previous
import functools

import jax
import jax.numpy as jnp
from jax import lax
from jax.experimental import pallas as pl
from jax.experimental.pallas import tpu as pltpu
from jax.experimental.pallas import tpu_sc as plsc

NN = 10000
NE = 320000
D = 128
H1, H2 = 512, 256

NCORE = 2
NSUB = 16
NW = NCORE * NSUB
EPW = NE // NW
CH = 80
NCHUNK = EPW // CH
NN_PAD = 10240
RPT = NN_PAD // NSUB
DEN_W = 16

BM = 400
GRID = NN // BM

_f32 = jnp.float32
_i32 = jnp.int32


def _relu6(v):
    return jnp.clip(v, 0.0, 6.0)


def _embed_body(x_ref, W1_ref, b1_ref, W2_ref, b2_ref, W3_ref, b3_ref,
                Wl_ref, bl_ref, Wr_ref, br_ref, xl_ref, xr_ref):
    h = jnp.dot(x_ref[...], W1_ref[...], preferred_element_type=_f32) + b1_ref[...]
    h = _relu6(h)
    h = jnp.dot(h, W2_ref[...], preferred_element_type=_f32) + b2_ref[...]
    h = _relu6(h)
    h = jnp.dot(h, W3_ref[...], preferred_element_type=_f32) + b3_ref[...]
    xl_ref[...] = jnp.dot(h, Wl_ref[...], preferred_element_type=_f32) + bl_ref[...]
    xr_ref[...] = jnp.dot(h, Wr_ref[...], preferred_element_type=_f32) + br_ref[...]


def _embed(x, W1, b1, W2, b2, W3, b3, Wl, bl, Wr, br):
    full = lambda a, b: pl.BlockSpec((a, b), lambda i: (0, 0))
    return pl.pallas_call(
        _embed_body,
        grid=(GRID,),
        in_specs=[
            pl.BlockSpec((BM, D), lambda i: (i, 0)),
            full(D, H1), full(1, H1),
            full(H1, H2), full(1, H2),
            full(H2, D), full(1, D),
            full(D, D), full(1, D),
            full(D, D), full(1, D),
        ],
        out_specs=[pl.BlockSpec((BM, D), lambda i: (i, 0)),
                   pl.BlockSpec((BM, D), lambda i: (i, 0))],
        out_shape=[jax.ShapeDtypeStruct((NN, D), _f32),
                   jax.ShapeDtypeStruct((NN, D), _f32)],
    )(x, W1, b1.reshape(1, -1), W2, b2.reshape(1, -1), W3, b3.reshape(1, -1),
      Wl, bl.reshape(1, -1), Wr, br.reshape(1, -1))


def _combine_proj_body(n_ref, d_ref, bias_ref, Wl_ref, bl_ref, Wr_ref, br_ref,
                       xl_ref, xr_ref):
    num = n_ref[0] + n_ref[1]
    den = jnp.sum(d_ref[...], axis=1)
    h = num / (den + 1e-16)[:, None] + bias_ref[...]
    h = _relu6(h)
    xl_ref[...] = jnp.dot(h, Wl_ref[...], preferred_element_type=_f32) + bl_ref[...]
    xr_ref[...] = jnp.dot(h, Wr_ref[...], preferred_element_type=_f32) + br_ref[...]


def _combine_proj(accN, accD, bias, Wl, bl, Wr, br):
    full = lambda a, b: pl.BlockSpec((a, b), lambda i: (0, 0))
    return pl.pallas_call(
        _combine_proj_body,
        grid=(GRID,),
        in_specs=[
            pl.BlockSpec((2, BM, D), lambda i: (0, i, 0)),
            pl.BlockSpec((BM, 2 * NSUB), lambda i: (i, 0)),
            full(1, D),
            full(D, D), full(1, D),
            full(D, D), full(1, D),
        ],
        out_specs=[pl.BlockSpec((BM, D), lambda i: (i, 0)),
                   pl.BlockSpec((BM, D), lambda i: (i, 0))],
        out_shape=[jax.ShapeDtypeStruct((NN, D), _f32),
                   jax.ShapeDtypeStruct((NN, D), _f32)],
    )(accN, accD, bias.reshape(1, -1), Wl, bl.reshape(1, -1), Wr, br.reshape(1, -1))


def _combine_final_body(n_ref, d_ref, bias_ref, out_ref):
    num = n_ref[0] + n_ref[1]
    den = jnp.sum(d_ref[...], axis=1)
    h = num / (den + 1e-16)[:, None] + bias_ref[...]
    out_ref[...] = _relu6(h)


def _combine_final(accN, accD, bias):
    full = lambda a, b: pl.BlockSpec((a, b), lambda i: (0, 0))
    return pl.pallas_call(
        _combine_final_body,
        grid=(GRID,),
        in_specs=[
            pl.BlockSpec((2, BM, D), lambda i: (0, i, 0)),
            pl.BlockSpec((BM, 2 * NSUB), lambda i: (i, 0)),
            full(1, D),
        ],
        out_specs=pl.BlockSpec((BM, D), lambda i: (i, 0)),
        out_shape=jax.ShapeDtypeStruct((NN, D), _f32),
    )(accN, accD, bias.reshape(1, -1))


def _gat_edge(xl, xr, src, dst, ea, We, att):
    mesh = plsc.VectorSubcoreMesh(core_axis_name="c", subcore_axis_name="s")

    @functools.partial(
        pl.kernel,
        mesh=mesh,
        compiler_params=pltpu.CompilerParams(needs_layout_passes=False),
        out_type=(jax.ShapeDtypeStruct((NCORE, NN_PAD, D), _f32),
                  jax.ShapeDtypeStruct((NCORE, NSUB, NN_PAD), _f32)),
        scratch_types=[
            pltpu.VMEM((2, CH), _i32),
            pltpu.VMEM((2, CH), _i32),
            pltpu.VMEM((2, CH), _f32),
            pltpu.VMEM((CH, D), _f32),
            pltpu.VMEM((CH, D), _f32),
            pltpu.VMEM((NN_PAD,), _f32),
            pltpu.VMEM((D,), _f32),
            pltpu.VMEM((D,), _f32),
            pltpu.VMEM_SHARED((NN_PAD, D), _f32),
            pltpu.SemaphoreType.DMA,
            pltpu.SemaphoreType.DMA,
        ],
    )
    def k(xl_h, xr_h, src_h, dst_h, ea_h, We_h, att_h, outN, outD,
          srcv2, dstv2, eav2, A, B, denv, Wev, attv, Nsh,
          semA, semB):
        c = lax.axis_index("c")
        s = lax.axis_index("s")
        w = c * NSUB + s

        pltpu.sync_copy(We_h, Wev)
        pltpu.sync_copy(att_h, attv)
        Wv = [Wev[16 * j:16 * (j + 1)] for j in range(D // 16)]
        Av = [attv[16 * j:16 * (j + 1)] for j in range(D // 16)]

        zero16 = jnp.zeros((16,), _f32)
        for e in range(CH):
            for j in range(D // 16):
                A[e, 16 * j:16 * (j + 1)] = zero16

        def zden(i, carry):
            denv[pl.ds(i * 16, 16)] = zero16
            return carry

        lax.fori_loop(0, NN_PAD // 16, zden, 0)
        r0 = s * RPT
        for r in range(RPT // CH):
            pltpu.sync_copy(A, Nsh.at[pl.ds(r0 + r * CH, CH)])

        def load_idx(i, row):
            base = w * EPW + i * CH
            pltpu.sync_copy(src_h.at[pl.ds(base, CH)], srcv2.at[row])
            pltpu.sync_copy(dst_h.at[pl.ds(base, CH)], dstv2.at[row])
            pltpu.sync_copy(ea_h.at[pl.ds(base, CH)], eav2.at[row])

        load_idx(0, 0)
        pltpu.async_copy(xl_h.at[srcv2.at[0]], A, semA)
        pltpu.async_copy(xr_h.at[dstv2.at[0]], B, semB)
        load_idx(1, 1)
        plsc.subcore_barrier()

        rows16 = lax.iota(_i32, 16)

        def chunk(i, carry):
            p = lax.rem(i, 2)
            q = 1 - p
            pltpu.make_async_copy(xl_h.at[srcv2.at[p]], A, semA).wait()
            pltpu.make_async_copy(xr_h.at[dstv2.at[p]], B, semB).wait()
            exvecs = []
            for g in range(CH // 16):
                ea_g = eav2[p, 16 * g:16 * (g + 1)]
                lvec = zero16
                for t in range(16):
                    e = g * 16 + t
                    ea_e = ea_g[t]
                    acc = zero16
                    for j in range(D // 16):
                        u = A[e, 16 * j:16 * (j + 1)] + B[e, 16 * j:16 * (j + 1)] + ea_e * Wv[j]
                        u = jnp.maximum(u, 0.2 * u)
                        acc = acc + Av[j] * u
                    lvec = jnp.where(rows16 == t, jnp.sum(acc), lvec)
                exvecs.append(jnp.exp(lvec))

            @pl.when(i < NCHUNK - 1)
            def _():
                pltpu.async_copy(xr_h.at[dstv2.at[q]], B, semB)

            for g in range(CH // 16):
                exvec = exvecs[g]
                plsc.addupdate_scatter(denv, [dstv2[p, 16 * g:16 * (g + 1)]], exvec)
                for t in range(16):
                    e = g * 16 + t
                    xs = exvec[t]
                    for j in range(D // 16):
                        A[e, 16 * j:16 * (j + 1)] = A[e, 16 * j:16 * (j + 1)] * xs
            pltpu.sync_copy(A, Nsh.at[dstv2.at[p]], add=True)

            @pl.when(i < NCHUNK - 1)
            def _():
                pltpu.async_copy(xl_h.at[srcv2.at[q]], A, semA)

            @pl.when(i < NCHUNK - 2)
            def _():
                load_idx(i + 2, p)

            return carry

        lax.fori_loop(0, NCHUNK, chunk, 0)
        pltpu.sync_copy(denv, outD.at[c, s])
        plsc.subcore_barrier()

        for r in range(RPT // CH):
            pltpu.sync_copy(Nsh.at[pl.ds(r0 + r * CH, CH)], A)
            pltpu.sync_copy(A, outN.at[c, pl.ds(r0 + r * CH, CH)])

    return k(xl, xr, src, dst, ea, We, att)


def kernel(x, edge_index, edge_attr, W1, b1, W2, b2, W3, b3,
           g0_Wl, g0_bl, g0_Wr, g0_br, g0_att, g0_We, g0_bias,
           g1_Wl, g1_bl, g1_Wr, g1_br, g1_att, g1_We, g1_bias):
    src = edge_index[0].astype(_i32)
    dst = edge_index[1].astype(_i32)
    ea = edge_attr.reshape(NE)

    xl0, xr0 = _embed(x, W1, b1, W2, b2, W3, b3, g0_Wl, g0_bl, g0_Wr, g0_br)
    N0, D0 = _gat_edge(xl0, xr0, src, dst, ea, g0_We.reshape(D), g0_att)
    d0 = D0.reshape(2 * NSUB, NN_PAD)[:, :NN].T
    xl1, xr1 = _combine_proj(N0[:, :NN], d0, g0_bias,
                             g1_Wl, g1_bl, g1_Wr, g1_br)
    N1, D1 = _gat_edge(xl1, xr1, src, dst, ea, g1_We.reshape(D), g1_att)
    d1 = D1.reshape(2 * NSUB, NN_PAD)[:, :NN].T
    return _combine_final(N1[:, :NN], d1, g1_bias)


# speedup vs baseline: 7.9004x; 1.0083x over previous
import functools

import jax
import jax.numpy as jnp
from jax import lax
from jax.experimental import pallas as pl
from jax.experimental.pallas import tpu as pltpu
from jax.experimental.pallas import tpu_sc as plsc

NN = 10000
NE = 320000
D = 128
H1, H2 = 512, 256

NCORE = 2
NSUB = 16
NW = NCORE * NSUB
EPW = NE // NW
CH = 80
NCHUNK = EPW // CH
NN_PAD = 10112
RPT = NN_PAD // NSUB
RFULL = RPT // CH
RREM = RPT - RFULL * CH
DEN_W = 16

BM = 400
GRID = NN // BM

_f32 = jnp.float32
_i32 = jnp.int32


def _relu6(v):
    return jnp.clip(v, 0.0, 6.0)


def _embed_body(x_ref, W1_ref, b1_ref, W2_ref, b2_ref, W3_ref, b3_ref,
                Wl_ref, bl_ref, Wr_ref, br_ref, xl_ref, xr_ref):
    h = jnp.dot(x_ref[...], W1_ref[...], preferred_element_type=_f32) + b1_ref[...]
    h = _relu6(h)
    h = jnp.dot(h, W2_ref[...], preferred_element_type=_f32) + b2_ref[...]
    h = _relu6(h)
    h = jnp.dot(h, W3_ref[...], preferred_element_type=_f32) + b3_ref[...]
    xl_ref[...] = jnp.dot(h, Wl_ref[...], preferred_element_type=_f32) + bl_ref[...]
    xr_ref[...] = jnp.dot(h, Wr_ref[...], preferred_element_type=_f32) + br_ref[...]


def _embed(x, W1, b1, W2, b2, W3, b3, Wl, bl, Wr, br):
    full = lambda a, b: pl.BlockSpec((a, b), lambda i: (0, 0))
    return pl.pallas_call(
        _embed_body,
        grid=(GRID,),
        in_specs=[
            pl.BlockSpec((BM, D), lambda i: (i, 0)),
            full(D, H1), full(1, H1),
            full(H1, H2), full(1, H2),
            full(H2, D), full(1, D),
            full(D, D), full(1, D),
            full(D, D), full(1, D),
        ],
        out_specs=[pl.BlockSpec((BM, D), lambda i: (i, 0)),
                   pl.BlockSpec((BM, D), lambda i: (i, 0))],
        out_shape=[jax.ShapeDtypeStruct((NN, D), _f32),
                   jax.ShapeDtypeStruct((NN, D), _f32)],
    )(x, W1, b1.reshape(1, -1), W2, b2.reshape(1, -1), W3, b3.reshape(1, -1),
      Wl, bl.reshape(1, -1), Wr, br.reshape(1, -1))


def _combine_proj_body(n_ref, d_ref, bias_ref, Wl_ref, bl_ref, Wr_ref, br_ref,
                       xl_ref, xr_ref):
    num = n_ref[0] + n_ref[1]
    den = jnp.sum(d_ref[...], axis=1)
    h = num / (den + 1e-16)[:, None] + bias_ref[...]
    h = _relu6(h)
    xl_ref[...] = jnp.dot(h, Wl_ref[...], preferred_element_type=_f32) + bl_ref[...]
    xr_ref[...] = jnp.dot(h, Wr_ref[...], preferred_element_type=_f32) + br_ref[...]


def _combine_proj(accN, accD, bias, Wl, bl, Wr, br):
    full = lambda a, b: pl.BlockSpec((a, b), lambda i: (0, 0))
    return pl.pallas_call(
        _combine_proj_body,
        grid=(GRID,),
        in_specs=[
            pl.BlockSpec((2, BM, D), lambda i: (0, i, 0)),
            pl.BlockSpec((BM, 2 * NSUB), lambda i: (i, 0)),
            full(1, D),
            full(D, D), full(1, D),
            full(D, D), full(1, D),
        ],
        out_specs=[pl.BlockSpec((BM, D), lambda i: (i, 0)),
                   pl.BlockSpec((BM, D), lambda i: (i, 0))],
        out_shape=[jax.ShapeDtypeStruct((NN, D), _f32),
                   jax.ShapeDtypeStruct((NN, D), _f32)],
    )(accN, accD, bias.reshape(1, -1), Wl, bl.reshape(1, -1), Wr, br.reshape(1, -1))


def _combine_final_body(n_ref, d_ref, bias_ref, out_ref):
    num = n_ref[0] + n_ref[1]
    den = jnp.sum(d_ref[...], axis=1)
    h = num / (den + 1e-16)[:, None] + bias_ref[...]
    out_ref[...] = _relu6(h)


def _combine_final(accN, accD, bias):
    full = lambda a, b: pl.BlockSpec((a, b), lambda i: (0, 0))
    return pl.pallas_call(
        _combine_final_body,
        grid=(GRID,),
        in_specs=[
            pl.BlockSpec((2, BM, D), lambda i: (0, i, 0)),
            pl.BlockSpec((BM, 2 * NSUB), lambda i: (i, 0)),
            full(1, D),
        ],
        out_specs=pl.BlockSpec((BM, D), lambda i: (i, 0)),
        out_shape=jax.ShapeDtypeStruct((NN, D), _f32),
    )(accN, accD, bias.reshape(1, -1))


def _gat_edge(xl, xr, src, dst, ea, We, att):
    mesh = plsc.VectorSubcoreMesh(core_axis_name="c", subcore_axis_name="s")

    @functools.partial(
        pl.kernel,
        mesh=mesh,
        compiler_params=pltpu.CompilerParams(needs_layout_passes=False),
        out_type=(jax.ShapeDtypeStruct((NCORE, NN_PAD, D), _f32),
                  jax.ShapeDtypeStruct((NCORE, NSUB, NN_PAD), _f32)),
        scratch_types=[
            pltpu.VMEM((2, CH), _i32),
            pltpu.VMEM((2, CH), _i32),
            pltpu.VMEM((2, CH), _f32),
            pltpu.VMEM((2, CH, D), _f32),
            pltpu.VMEM((CH, D), _f32),
            pltpu.VMEM((NN_PAD,), _f32),
            pltpu.VMEM((D,), _f32),
            pltpu.VMEM((D,), _f32),
            pltpu.VMEM_SHARED((NN_PAD, D), _f32),
            pltpu.SemaphoreType.DMA,
            pltpu.SemaphoreType.DMA,
        ],
    )
    def k(xl_h, xr_h, src_h, dst_h, ea_h, We_h, att_h, outN, outD,
          srcv2, dstv2, eav2, A2, B, denv, Wev, attv, Nsh,
          semA, semB):
        c = lax.axis_index("c")
        s = lax.axis_index("s")
        w = c * NSUB + s

        pltpu.sync_copy(We_h, Wev)
        pltpu.sync_copy(att_h, attv)
        Wv = [Wev[16 * j:16 * (j + 1)] for j in range(D // 16)]
        Av = [attv[16 * j:16 * (j + 1)] for j in range(D // 16)]

        zero16 = jnp.zeros((16,), _f32)
        for e in range(CH):
            for j in range(D // 16):
                A2[0, e, 16 * j:16 * (j + 1)] = zero16

        def zden(i, carry):
            denv[pl.ds(i * 16, 16)] = zero16
            return carry

        lax.fori_loop(0, NN_PAD // 16, zden, 0)
        r0 = s * RPT
        for r in range(RFULL):
            pltpu.sync_copy(A2.at[0], Nsh.at[pl.ds(r0 + r * CH, CH)])
        pltpu.sync_copy(A2.at[0, pl.ds(0, RREM)],
                        Nsh.at[pl.ds(r0 + RFULL * CH, RREM)])

        def load_idx(i, row):
            base = w * EPW + i * CH
            pltpu.sync_copy(src_h.at[pl.ds(base, CH)], srcv2.at[row])
            pltpu.sync_copy(dst_h.at[pl.ds(base, CH)], dstv2.at[row])
            pltpu.sync_copy(ea_h.at[pl.ds(base, CH)], eav2.at[row])

        load_idx(0, 0)
        pltpu.async_copy(xl_h.at[srcv2.at[0]], A2.at[0], semA)
        pltpu.async_copy(xr_h.at[dstv2.at[0]], B, semB)
        load_idx(1, 1)
        plsc.subcore_barrier()

        rows16 = lax.iota(_i32, 16)

        def chunk(i, carry):
            p = lax.rem(i, 2)
            q = 1 - p
            pltpu.make_async_copy(xl_h.at[srcv2.at[p]], A2.at[p], semA).wait()
            pltpu.make_async_copy(xr_h.at[dstv2.at[p]], B, semB).wait()

            @pl.when(i < NCHUNK - 1)
            def _():
                pltpu.async_copy(xl_h.at[srcv2.at[q]], A2.at[q], semA)

            exvecs = []
            for g in range(CH // 16):
                ea_g = eav2[p, 16 * g:16 * (g + 1)]
                lvec = zero16
                for t in range(16):
                    e = g * 16 + t
                    ea_e = ea_g[t]
                    acc = zero16
                    for j in range(D // 16):
                        u = A2[p, e, 16 * j:16 * (j + 1)] + B[e, 16 * j:16 * (j + 1)] + ea_e * Wv[j]
                        u = jnp.maximum(u, 0.2 * u)
                        acc = acc + Av[j] * u
                    lvec = jnp.where(rows16 == t, jnp.sum(acc), lvec)
                exvecs.append(jnp.exp(lvec))

            @pl.when(i < NCHUNK - 1)
            def _():
                pltpu.async_copy(xr_h.at[dstv2.at[q]], B, semB)

            for g in range(CH // 16):
                exvec = exvecs[g]
                plsc.addupdate_scatter(denv, [dstv2[p, 16 * g:16 * (g + 1)]], exvec)
                for t in range(16):
                    e = g * 16 + t
                    xs = exvec[t]
                    for j in range(D // 16):
                        A2[p, e, 16 * j:16 * (j + 1)] = A2[p, e, 16 * j:16 * (j + 1)] * xs
            pltpu.sync_copy(A2.at[p], Nsh.at[dstv2.at[p]], add=True)

            @pl.when(i < NCHUNK - 2)
            def _():
                load_idx(i + 2, p)

            return carry

        lax.fori_loop(0, NCHUNK, chunk, 0)
        pltpu.sync_copy(denv, outD.at[c, s])
        plsc.subcore_barrier()

        for r in range(RFULL):
            pltpu.sync_copy(Nsh.at[pl.ds(r0 + r * CH, CH)], A2.at[0])
            pltpu.sync_copy(A2.at[0], outN.at[c, pl.ds(r0 + r * CH, CH)])
        pltpu.sync_copy(Nsh.at[pl.ds(r0 + RFULL * CH, RREM)],
                        A2.at[0, pl.ds(0, RREM)])
        pltpu.sync_copy(A2.at[0, pl.ds(0, RREM)],
                        outN.at[c, pl.ds(r0 + RFULL * CH, RREM)])

    return k(xl, xr, src, dst, ea, We, att)


def kernel(x, edge_index, edge_attr, W1, b1, W2, b2, W3, b3,
           g0_Wl, g0_bl, g0_Wr, g0_br, g0_att, g0_We, g0_bias,
           g1_Wl, g1_bl, g1_Wr, g1_br, g1_att, g1_We, g1_bias):
    src = edge_index[0].astype(_i32)
    dst = edge_index[1].astype(_i32)
    ea = edge_attr.reshape(NE)

    xl0, xr0 = _embed(x, W1, b1, W2, b2, W3, b3, g0_Wl, g0_bl, g0_Wr, g0_br)
    N0, D0 = _gat_edge(xl0, xr0, src, dst, ea, g0_We.reshape(D), g0_att)
    d0 = D0.reshape(2 * NSUB, NN_PAD)[:, :NN].T
    xl1, xr1 = _combine_proj(N0[:, :NN], d0, g0_bias,
                             g1_Wl, g1_bl, g1_Wr, g1_br)
    N1, D1 = _gat_edge(xl1, xr1, src, dst, ea, g1_We.reshape(D), g1_att)
    d1 = D1.reshape(2 * NSUB, NN_PAD)[:, :NN].T
    return _combine_final(N1[:, :NN], d1, g1_bias)
